# Initial kernel scaffold; baseline (speedup 1.0000x reference)
import jax
import jax.numpy as jnp
from jax.experimental import pallas as pl


def kernel(x, edge_index, graph_ids, target_idx, W1, b1, W2, b2, Wout, bout):
    raise NotImplementedError("write your pallas kernel here")


# speedup vs baseline: 5.3294x; 5.3294x over previous
import functools

import jax
import jax.numpy as jnp
from jax import lax
from jax.experimental import pallas as pl
from jax.experimental.pallas import tpu as pltpu
from jax.experimental.pallas import tpu_sc as plsc

_NC = 2
_NS = 16
_NW = _NC * _NS
_K = 128


def _sc_segment_sum(h, srcm, dstm, zrows):
    n, d = h.shape
    _, ch, _, k = srcm.shape
    nr = zrows.shape[0]
    zr = nr // _NS

    mesh = plsc.VectorSubcoreMesh(core_axis_name="c", subcore_axis_name="s")

    @functools.partial(
        pl.kernel,
        mesh=mesh,
        out_type=jax.ShapeDtypeStruct((_NC, nr, d), jnp.float32),
        scratch_types=[
            pltpu.VMEM((ch, k), jnp.int32),
            pltpu.VMEM((1, k), jnp.int32),
            pltpu.VMEM((1, k), jnp.int32),
            pltpu.VMEM((k, d), jnp.float32),
            pltpu.VMEM((k, d), jnp.float32),
            pltpu.VMEM_SHARED((nr, d), jnp.float32),
            pltpu.SemaphoreType.DMA,
            pltpu.SemaphoreType.DMA,
            pltpu.SemaphoreType.DMA,
            pltpu.SemaphoreType.DMA,
        ],
    )
    def agg(h_hbm, srcm_hbm, dstm_hbm, z_hbm, out_hbm,
            didx, sb0, sb1, rows0, rows1, acc, sem_a, sem_b, sem_i0, sem_i1):
        c = lax.axis_index("c")
        s = lax.axis_index("s")
        wid = c * _NS + s
        pltpu.sync_copy(z_hbm.at[pl.ds(s * zr, zr)], acc.at[pl.ds(s * zr, zr)])
        pltpu.sync_copy(dstm_hbm.at[wid], didx)
        plsc.subcore_barrier()

        def start_idx(j, buf, sem):
            pltpu.make_async_copy(srcm_hbm.at[wid, j], buf, sem).start()

        def wait_idx(j, buf, sem):
            pltpu.make_async_copy(srcm_hbm.at[wid, j], buf, sem).wait()

        def start(buf_i, buf, sem):
            pltpu.make_async_copy(h_hbm.at[buf_i.at[0]], buf, sem).start()

        def wait(buf_i, buf, sem):
            pltpu.make_async_copy(h_hbm.at[buf_i.at[0]], buf, sem).wait()

        def scat(j, buf):
            pltpu.sync_copy(buf, acc.at[didx.at[j]], add=True)

        start_idx(0, sb0, sem_i0)
        wait_idx(0, sb0, sem_i0)
        start(sb0, rows0, sem_a)
        if ch > 1:
            start_idx(1, sb1, sem_i1)

        def pair(p, carry):
            j0 = 2 * p
            wait_idx(j0 + 1, sb1, sem_i1)
            start(sb1, rows1, sem_b)
            wait(sb0, rows0, sem_a)
            scat(j0, rows0)
            start_idx(j0 + 2, sb0, sem_i0)
            wait_idx(j0 + 2, sb0, sem_i0)
            start(sb0, rows0, sem_a)
            wait(sb1, rows1, sem_b)
            scat(j0 + 1, rows1)

            @pl.when(j0 + 3 < ch)
            def _():
                start_idx(j0 + 3, sb1, sem_i1)

            return carry

        lax.fori_loop(0, (ch - 1) // 2, pair, 0)
        wait(sb0, rows0, sem_a)
        scat(ch - 1, rows0)

        plsc.subcore_barrier()
        pltpu.sync_copy(acc.at[pl.ds(s * zr, zr)],
                        out_hbm.at[c, pl.ds(s * zr, zr)])

    return agg(h, srcm, dstm, zrows)


def _tc_linear(parts, w, b2d, rb):
    _, n, d = parts.shape
    nblk = n // rb

    def body(p_ref, w_ref, b_ref, o_ref):
        a = (p_ref[0] + p_ref[1]).astype(jnp.bfloat16)
        wb = w_ref[...].astype(jnp.bfloat16)
        o_ref[...] = jnp.maximum(
            jnp.dot(a, wb, preferred_element_type=jnp.float32)
            + b_ref[...], 0.0)

    return pl.pallas_call(
        body,
        grid=(nblk,),
        in_specs=[
            pl.BlockSpec((2, rb, d), lambda i: (0, i, 0)),
            pl.BlockSpec((d, d), lambda i: (0, 0)),
            pl.BlockSpec((1, d), lambda i: (0, 0)),
        ],
        out_specs=pl.BlockSpec((rb, d), lambda i: (i, 0)),
        out_shape=jax.ShapeDtypeStruct((n, d), jnp.float32),
    )(parts, w, b2d)


def _tc_readout(node_embed, gid3, tid2, wout2, bout2, rb):
    n, d = node_embed.shape
    nblk = n // rb
    b = tid2.shape[0]

    def body(ne_ref, gid_ref, tid_ref, w_ref, bo_ref, o_ref,
             sums, tsums, counts):
        i = pl.program_id(0)

        @pl.when(i == 0)
        def _():
            sums[...] = jnp.zeros_like(sums)
            tsums[...] = jnp.zeros_like(tsums)
            counts[...] = jnp.zeros_like(counts)

        rows = ne_ref[...]
        ids = gid_ref[0]
        onehot = (lax.broadcasted_iota(jnp.int32, (b, rb), 0)
                  == ids).astype(jnp.float32)
        sums[...] += jnp.dot(onehot, rows, preferred_element_type=jnp.float32,
                           precision=lax.Precision.HIGHEST)
        counts[...] += jnp.sum(onehot, axis=1, keepdims=True)
        rowids = i * rb + lax.broadcasted_iota(jnp.int32, (b, rb), 1)
        tmask = (tid_ref[...] == rowids).astype(jnp.float32)
        tsums[...] += jnp.dot(tmask, rows, preferred_element_type=jnp.float32,
                            precision=lax.Precision.HIGHEST)

        @pl.when(i == pl.num_programs(0) - 1)
        def _():
            ge = sums[...] / jnp.maximum(counts[...], 1.0)
            ge = ge.astype(jnp.bfloat16).astype(jnp.float32)
            te = tsums[...].astype(jnp.bfloat16).astype(jnp.float32)
            wb = w_ref[...].astype(jnp.bfloat16).astype(jnp.float32)
            logits = (jnp.sum(ge * wb[0:1, :], axis=1, keepdims=True)
                      + jnp.sum(te * wb[1:2, :], axis=1,
                                keepdims=True)
                      + bo_ref[...])
            o_ref[...] = 1.0 / (1.0 + jnp.exp(-logits))

    return pl.pallas_call(
        body,
        grid=(nblk,),
        in_specs=[
            pl.BlockSpec((rb, d), lambda i: (i, 0)),
            pl.BlockSpec((1, 1, rb), lambda i: (i, 0, 0)),
            pl.BlockSpec((b, 1), lambda i: (0, 0)),
            pl.BlockSpec((2, d), lambda i: (0, 0)),
            pl.BlockSpec((1, 1), lambda i: (0, 0)),
        ],
        out_specs=pl.BlockSpec((b, 1), lambda i: (0, 0)),
        out_shape=jax.ShapeDtypeStruct((b, 1), jnp.float32),
        scratch_shapes=[
            pltpu.VMEM((b, d), jnp.float32),
            pltpu.VMEM((b, d), jnp.float32),
            pltpu.VMEM((b, 1), jnp.float32),
        ],
    )(node_embed, gid3, tid2, wout2, bout2)


def kernel(x, edge_index, graph_ids, target_idx, W1, b1, W2, b2, Wout, bout):
    n, d = x.shape
    e = edge_index.shape[1]
    b = target_idx.shape[0]

    ch = -(-e // (_NW * _K))
    if ch % 2 == 0:
        ch += 1
    pad = _NW * ch * _K - e
    src = jnp.concatenate([edge_index[0], jnp.zeros((pad,), jnp.int32)])
    dst = jnp.concatenate([edge_index[1], jnp.full((pad,), n, jnp.int32)])
    srcm = src.reshape(_NW, ch, 1, _K)
    dstm = dst.reshape(_NW, ch, _K)
    nr = -(-(n + 1) // (_NS * 8)) * (_NS * 8)
    zrows = jnp.zeros((nr, d), jnp.float32)

    b1r = b1.reshape(1, d)
    b2r = b2.reshape(1, d)
    wout2 = Wout.reshape(2, d)
    bout2 = bout.reshape(1, 1)
    rb = 1000
    gid3 = graph_ids.reshape(n // rb, 1, rb)
    tid2 = target_idx.reshape(b, 1)

    p1 = _sc_segment_sum(x, srcm, dstm, zrows)
    h1 = _tc_linear(p1, W1, b1r, rb)
    p2 = _sc_segment_sum(h1, srcm, dstm, zrows)
    h2 = _tc_linear(p2, W2, b2r, rb)
    return _tc_readout(h2, gid3, tid2, wout2, bout2, rb)


# speedup vs baseline: 6.2244x; 1.1679x over previous
import functools

import jax
import jax.numpy as jnp
from jax import lax
from jax.experimental import pallas as pl
from jax.experimental.pallas import tpu as pltpu
from jax.experimental.pallas import tpu_sc as plsc

_NC = 2
_NS = 16
_NW = _NC * _NS
_K = 128


def _sc_segment_sum(h, srcm, dstm, zrows):
    n, d = h.shape
    _, ch, _, k = srcm.shape
    nr = zrows.shape[0]
    zr = nr // _NS

    mesh = plsc.VectorSubcoreMesh(core_axis_name="c", subcore_axis_name="s")

    @functools.partial(
        pl.kernel,
        mesh=mesh,
        out_type=jax.ShapeDtypeStruct((_NC, nr, d), jnp.float32),
        scratch_types=[
            pltpu.VMEM((1, k), jnp.int32),
            pltpu.VMEM((1, k), jnp.int32),
            pltpu.VMEM((1, k), jnp.int32),
            pltpu.VMEM((1, k), jnp.int32),
            pltpu.VMEM((1, k), jnp.int32),
            pltpu.VMEM((1, k), jnp.int32),
            pltpu.VMEM((k, d), jnp.float32),
            pltpu.VMEM((k, d), jnp.float32),
            pltpu.VMEM((k, d), jnp.float32),
            pltpu.VMEM_SHARED((nr, d), jnp.float32),
        ] + [pltpu.SemaphoreType.DMA] * 12,
    )
    def agg(h_hbm, srcm_hbm, dstm_hbm, z_hbm, out_hbm,
            s0, s1, s2, t0, t1, t2, r0, r1, r2, acc, *sems):
        c = lax.axis_index("c")
        s = lax.axis_index("s")
        wid = c * _NS + s
        sb = (s0, s1, s2)
        tb = (t0, t1, t2)
        rb_ = (r0, r1, r2)
        gsem = sems[0:3]
        ssem = sems[3:6]
        sisem = sems[6:9]
        tisem = sems[9:12]
        pltpu.sync_copy(z_hbm.at[pl.ds(s * zr, zr)], acc.at[pl.ds(s * zr, zr)])
        plsc.subcore_barrier()

        def start_sidx(j, b):
            pltpu.make_async_copy(srcm_hbm.at[wid, j], sb[b], sisem[b]).start()

        def wait_sidx(j, b):
            pltpu.make_async_copy(srcm_hbm.at[wid, j], sb[b], sisem[b]).wait()

        def start_tidx(j, b):
            pltpu.make_async_copy(dstm_hbm.at[wid, j], tb[b], tisem[b]).start()

        def wait_tidx(j, b):
            pltpu.make_async_copy(dstm_hbm.at[wid, j], tb[b], tisem[b]).wait()

        def start_g(b):
            pltpu.make_async_copy(h_hbm.at[sb[b].at[0]], rb_[b], gsem[b]).start()

        def wait_g(b):
            pltpu.make_async_copy(h_hbm.at[sb[b].at[0]], rb_[b], gsem[b]).wait()

        def start_s(b):
            pltpu.async_copy(rb_[b], acc.at[tb[b].at[0]], ssem[b], add=True)

        def wait_s(b):
            pltpu.make_async_copy(rb_[b], acc.at[tb[b].at[0]], ssem[b]).wait()

        start_sidx(0, 0)
        if ch > 1:
            start_sidx(1, 1)
        start_tidx(0, 0)

        def step(q3, j):
            b = j % 3
            bm1 = (j - 1) % 3
            bm2 = (j - 2) % 3
            jt = q3 + j

            @pl.when((jt >= 2) & (jt - 2 < ch))
            def _():
                wait_s(bm2)

            @pl.when(jt + 1 < ch)
            def _():
                start_tidx(jt + 1, (j + 1) % 3)

            @pl.when(jt < ch)
            def _():
                wait_sidx(jt, b)
                start_g(b)

            @pl.when((jt >= 1) & (jt - 1 < ch))
            def _():
                wait_g(bm1)
                wait_tidx(jt - 1, bm1)
                start_s(bm1)

            @pl.when(jt + 2 < ch)
            def _():
                start_sidx(jt + 2, (j + 2) % 3)

        n_outer = -(-(ch + 2) // 3)

        def outer(q, carry):
            q3 = 3 * q
            step(q3, 0)
            step(q3, 1)
            step(q3, 2)
            return carry

        lax.fori_loop(0, n_outer, outer, 0)

        plsc.subcore_barrier()
        pltpu.sync_copy(acc.at[pl.ds(s * zr, zr)],
                        out_hbm.at[c, pl.ds(s * zr, zr)])

    return agg(h, srcm, dstm, zrows)


def _tc_linear(parts, w, b2d, rb):
    _, n, d = parts.shape
    nblk = n // rb

    def body(p_ref, w_ref, b_ref, o_ref):
        a = (p_ref[0] + p_ref[1]).astype(jnp.bfloat16)
        wb = w_ref[...].astype(jnp.bfloat16)
        o_ref[...] = jnp.maximum(
            jnp.dot(a, wb, preferred_element_type=jnp.float32)
            + b_ref[...], 0.0)

    return pl.pallas_call(
        body,
        grid=(nblk,),
        in_specs=[
            pl.BlockSpec((2, rb, d), lambda i: (0, i, 0)),
            pl.BlockSpec((d, d), lambda i: (0, 0)),
            pl.BlockSpec((1, d), lambda i: (0, 0)),
        ],
        out_specs=pl.BlockSpec((rb, d), lambda i: (i, 0)),
        out_shape=jax.ShapeDtypeStruct((n, d), jnp.float32),
    )(parts, w, b2d)


def _tc_readout(node_embed, gid3, tid2, wout2, bout2, rb):
    n, d = node_embed.shape
    nblk = n // rb
    b = tid2.shape[0]

    def body(ne_ref, gid_ref, tid_ref, w_ref, bo_ref, o_ref,
             sums, tsums, counts):
        i = pl.program_id(0)

        @pl.when(i == 0)
        def _():
            sums[...] = jnp.zeros_like(sums)
            tsums[...] = jnp.zeros_like(tsums)
            counts[...] = jnp.zeros_like(counts)

        rows = ne_ref[...]
        ids = gid_ref[0]
        onehot = (lax.broadcasted_iota(jnp.int32, (b, rb), 0)
                  == ids).astype(jnp.float32)
        sums[...] += jnp.dot(onehot, rows, preferred_element_type=jnp.float32,
                           precision=lax.Precision.HIGHEST)
        counts[...] += jnp.sum(onehot, axis=1, keepdims=True)
        rowids = i * rb + lax.broadcasted_iota(jnp.int32, (b, rb), 1)
        tmask = (tid_ref[...] == rowids).astype(jnp.float32)
        tsums[...] += jnp.dot(tmask, rows, preferred_element_type=jnp.float32,
                            precision=lax.Precision.HIGHEST)

        @pl.when(i == pl.num_programs(0) - 1)
        def _():
            ge = sums[...] / jnp.maximum(counts[...], 1.0)
            ge = ge.astype(jnp.bfloat16).astype(jnp.float32)
            te = tsums[...].astype(jnp.bfloat16).astype(jnp.float32)
            wb = w_ref[...].astype(jnp.bfloat16).astype(jnp.float32)
            logits = (jnp.sum(ge * wb[0:1, :], axis=1, keepdims=True)
                      + jnp.sum(te * wb[1:2, :], axis=1,
                                keepdims=True)
                      + bo_ref[...])
            o_ref[...] = 1.0 / (1.0 + jnp.exp(-logits))

    return pl.pallas_call(
        body,
        grid=(nblk,),
        in_specs=[
            pl.BlockSpec((rb, d), lambda i: (i, 0)),
            pl.BlockSpec((1, 1, rb), lambda i: (i, 0, 0)),
            pl.BlockSpec((b, 1), lambda i: (0, 0)),
            pl.BlockSpec((2, d), lambda i: (0, 0)),
            pl.BlockSpec((1, 1), lambda i: (0, 0)),
        ],
        out_specs=pl.BlockSpec((b, 1), lambda i: (0, 0)),
        out_shape=jax.ShapeDtypeStruct((b, 1), jnp.float32),
        scratch_shapes=[
            pltpu.VMEM((b, d), jnp.float32),
            pltpu.VMEM((b, d), jnp.float32),
            pltpu.VMEM((b, 1), jnp.float32),
        ],
    )(node_embed, gid3, tid2, wout2, bout2)


def kernel(x, edge_index, graph_ids, target_idx, W1, b1, W2, b2, Wout, bout):
    n, d = x.shape
    e = edge_index.shape[1]
    b = target_idx.shape[0]

    ch = -(-e // (_NW * _K))
    if ch % 2 == 0:
        ch += 1
    pad = _NW * ch * _K - e
    src = jnp.concatenate([edge_index[0], jnp.zeros((pad,), jnp.int32)])
    dst = jnp.concatenate([edge_index[1], jnp.full((pad,), n, jnp.int32)])
    srcm = src.reshape(_NW, ch, 1, _K)
    dstm = dst.reshape(_NW, ch, 1, _K)
    nr = -(-(n + 1) // (_NS * 8)) * (_NS * 8)
    zrows = jnp.zeros((nr, d), jnp.float32)

    b1r = b1.reshape(1, d)
    b2r = b2.reshape(1, d)
    wout2 = Wout.reshape(2, d)
    bout2 = bout.reshape(1, 1)
    rb = 1000
    gid3 = graph_ids.reshape(n // rb, 1, rb)
    tid2 = target_idx.reshape(b, 1)

    p1 = _sc_segment_sum(x, srcm, dstm, zrows)
    h1 = _tc_linear(p1, W1, b1r, rb)
    p2 = _sc_segment_sum(h1, srcm, dstm, zrows)
    h2 = _tc_linear(p2, W2, b2r, rb)
    return _tc_readout(h2, gid3, tid2, wout2, bout2, rb)


# speedup vs baseline: 6.3116x; 1.0140x over previous
import functools

import jax
import jax.numpy as jnp
from jax import lax
from jax.experimental import pallas as pl
from jax.experimental.pallas import tpu as pltpu
from jax.experimental.pallas import tpu_sc as plsc

_NC = 2
_NS = 16
_NW = _NC * _NS
_K = 128


def _sc_segment_sum(h, srcm, dstm, zrows):
    n, d = h.shape
    _, ch, _, k = srcm.shape
    nr = zrows.shape[0]
    zr = nr // _NS

    mesh = plsc.VectorSubcoreMesh(core_axis_name="c", subcore_axis_name="s")

    @functools.partial(
        pl.kernel,
        mesh=mesh,
        out_type=jax.ShapeDtypeStruct((_NC, nr, d), jnp.float32),
        scratch_types=[
            pltpu.VMEM((1, k), jnp.int32),
            pltpu.VMEM((1, k), jnp.int32),
            pltpu.VMEM((1, k), jnp.int32),
            pltpu.VMEM((1, k), jnp.int32),
            pltpu.VMEM((1, k), jnp.int32),
            pltpu.VMEM((1, k), jnp.int32),
            pltpu.VMEM((k, d), jnp.float32),
            pltpu.VMEM((k, d), jnp.float32),
            pltpu.VMEM((k, d), jnp.float32),
            pltpu.VMEM_SHARED((nr, d), jnp.float32),
        ] + [pltpu.SemaphoreType.DMA] * 12,
    )
    def agg(h_hbm, srcm_hbm, dstm_hbm, z_hbm, out_hbm,
            s0, s1, s2, t0, t1, t2, r0, r1, r2, acc, *sems):
        c = lax.axis_index("c")
        s = lax.axis_index("s")
        wid = c * _NS + s
        sb = (s0, s1, s2)
        tb = (t0, t1, t2)
        rb_ = (r0, r1, r2)
        gsem = sems[0:3]
        ssem = sems[3:6]
        sisem = sems[6:9]
        tisem = sems[9:12]
        pltpu.sync_copy(z_hbm.at[pl.ds(s * zr, zr)], acc.at[pl.ds(s * zr, zr)])
        plsc.subcore_barrier()

        def start_sidx(j, b):
            pltpu.make_async_copy(srcm_hbm.at[wid, j], sb[b], sisem[b]).start()

        def wait_sidx(j, b):
            pltpu.make_async_copy(srcm_hbm.at[wid, j], sb[b], sisem[b]).wait()

        def start_tidx(j, b):
            pltpu.make_async_copy(dstm_hbm.at[wid, j], tb[b], tisem[b]).start()

        def wait_tidx(j, b):
            pltpu.make_async_copy(dstm_hbm.at[wid, j], tb[b], tisem[b]).wait()

        def start_g(b):
            pltpu.make_async_copy(h_hbm.at[sb[b].at[0]], rb_[b], gsem[b]).start()

        def wait_g(b):
            pltpu.make_async_copy(h_hbm.at[sb[b].at[0]], rb_[b], gsem[b]).wait()

        def start_s(b):
            pltpu.async_copy(rb_[b], acc.at[tb[b].at[0]], ssem[b], add=True)

        def wait_s(b):
            pltpu.make_async_copy(rb_[b], acc.at[tb[b].at[0]], ssem[b]).wait()

        start_sidx(0, 0)
        if ch > 1:
            start_sidx(1, 1)
        start_tidx(0, 0)

        def step(q3, j):
            b = j % 3
            bm1 = (j - 1) % 3
            bm2 = (j - 2) % 3
            jt = q3 + j

            @pl.when((jt >= 2) & (jt - 2 < ch))
            def _():
                wait_s(bm2)

            @pl.when(jt + 1 < ch)
            def _():
                start_tidx(jt + 1, (j + 1) % 3)

            @pl.when(jt < ch)
            def _():
                wait_sidx(jt, b)
                start_g(b)

            @pl.when((jt >= 1) & (jt - 1 < ch))
            def _():
                wait_g(bm1)
                wait_tidx(jt - 1, bm1)
                start_s(bm1)

            @pl.when(jt + 2 < ch)
            def _():
                start_sidx(jt + 2, (j + 2) % 3)

        n_outer = -(-(ch + 2) // 3)

        def outer(q, carry):
            q3 = 3 * q
            step(q3, 0)
            step(q3, 1)
            step(q3, 2)
            return carry

        lax.fori_loop(0, n_outer, outer, 0)

        plsc.subcore_barrier()
        pltpu.sync_copy(acc.at[pl.ds(s * zr, zr)],
                        out_hbm.at[c, pl.ds(s * zr, zr)])

    return agg(h, srcm, dstm, zrows)


def _tc_linear(parts, w, b2d, rb):
    _, n, d = parts.shape
    nblk = n // rb

    def body(p_ref, w_ref, b_ref, o_ref):
        a = (p_ref[0] + p_ref[1]).astype(jnp.bfloat16)
        wb = w_ref[...].astype(jnp.bfloat16)
        o_ref[...] = jnp.maximum(
            jnp.dot(a, wb, preferred_element_type=jnp.float32)
            + b_ref[...], 0.0)

    return pl.pallas_call(
        body,
        grid=(nblk,),
        in_specs=[
            pl.BlockSpec((2, rb, d), lambda i: (0, i, 0)),
            pl.BlockSpec((d, d), lambda i: (0, 0)),
            pl.BlockSpec((1, d), lambda i: (0, 0)),
        ],
        out_specs=pl.BlockSpec((rb, d), lambda i: (i, 0)),
        out_shape=jax.ShapeDtypeStruct((n, d), jnp.float32),
    )(parts, w, b2d)


def _tc_readout(parts, w2, b2d, gid3, tid2, wout2, bout2, rb, n):
    _, _, d = parts.shape
    nblk = n // rb
    b = tid2.shape[0]

    def body(p_ref, w2_ref, b2_ref, gid_ref, tid_ref, w_ref, bo_ref, o_ref,
             sums, tsums, counts):
        i = pl.program_id(0)

        @pl.when(i == 0)
        def _():
            sums[...] = jnp.zeros_like(sums)
            tsums[...] = jnp.zeros_like(tsums)
            counts[...] = jnp.zeros_like(counts)

        a = (p_ref[0] + p_ref[1]).astype(jnp.bfloat16)
        wb2 = w2_ref[...].astype(jnp.bfloat16)
        rows = jnp.maximum(
            jnp.dot(a, wb2, preferred_element_type=jnp.float32)
            + b2_ref[...], 0.0)
        ids = gid_ref[0]
        onehot = (lax.broadcasted_iota(jnp.int32, (b, rb), 0)
                  == ids).astype(jnp.float32)
        sums[...] += jnp.dot(onehot, rows, preferred_element_type=jnp.float32,
                           precision=lax.Precision.HIGHEST)
        counts[...] += jnp.sum(onehot, axis=1, keepdims=True)
        rowids = i * rb + lax.broadcasted_iota(jnp.int32, (b, rb), 1)
        tmask = (tid_ref[...] == rowids).astype(jnp.float32)
        tsums[...] += jnp.dot(tmask, rows, preferred_element_type=jnp.float32,
                            precision=lax.Precision.HIGHEST)

        @pl.when(i == pl.num_programs(0) - 1)
        def _():
            ge = sums[...] / jnp.maximum(counts[...], 1.0)
            ge = ge.astype(jnp.bfloat16).astype(jnp.float32)
            te = tsums[...].astype(jnp.bfloat16).astype(jnp.float32)
            wb = w_ref[...].astype(jnp.bfloat16).astype(jnp.float32)
            logits = (jnp.sum(ge * wb[0:1, :], axis=1, keepdims=True)
                      + jnp.sum(te * wb[1:2, :], axis=1,
                                keepdims=True)
                      + bo_ref[...])
            o_ref[...] = 1.0 / (1.0 + jnp.exp(-logits))

    return pl.pallas_call(
        body,
        grid=(nblk,),
        in_specs=[
            pl.BlockSpec((2, rb, d), lambda i: (0, i, 0)),
            pl.BlockSpec((d, d), lambda i: (0, 0)),
            pl.BlockSpec((1, d), lambda i: (0, 0)),
            pl.BlockSpec((1, 1, rb), lambda i: (i, 0, 0)),
            pl.BlockSpec((b, 1), lambda i: (0, 0)),
            pl.BlockSpec((2, d), lambda i: (0, 0)),
            pl.BlockSpec((1, 1), lambda i: (0, 0)),
        ],
        out_specs=pl.BlockSpec((b, 1), lambda i: (0, 0)),
        out_shape=jax.ShapeDtypeStruct((b, 1), jnp.float32),
        scratch_shapes=[
            pltpu.VMEM((b, d), jnp.float32),
            pltpu.VMEM((b, d), jnp.float32),
            pltpu.VMEM((b, 1), jnp.float32),
        ],
    )(parts, w2, b2d, gid3, tid2, wout2, bout2)


def kernel(x, edge_index, graph_ids, target_idx, W1, b1, W2, b2, Wout, bout):
    n, d = x.shape
    e = edge_index.shape[1]
    b = target_idx.shape[0]

    ch = -(-e // (_NW * _K))
    if ch % 2 == 0:
        ch += 1
    pad = _NW * ch * _K - e
    src = jnp.concatenate([edge_index[0], jnp.zeros((pad,), jnp.int32)])
    dst = jnp.concatenate([edge_index[1], jnp.full((pad,), n, jnp.int32)])
    srcm = src.reshape(_NW, ch, 1, _K)
    dstm = dst.reshape(_NW, ch, 1, _K)
    nr = -(-(n + 1) // (_NS * 8)) * (_NS * 8)
    zrows = jnp.zeros((nr, d), jnp.float32)

    b1r = b1.reshape(1, d)
    b2r = b2.reshape(1, d)
    wout2 = Wout.reshape(2, d)
    bout2 = bout.reshape(1, 1)
    rb = 1000
    gid3 = graph_ids.reshape(n // rb, 1, rb)
    tid2 = target_idx.reshape(b, 1)

    p1 = _sc_segment_sum(x, srcm, dstm, zrows)
    h1 = _tc_linear(p1, W1, b1r, rb)
    p2 = _sc_segment_sum(h1, srcm, dstm, zrows)
    return _tc_readout(p2, W2, b2r, gid3, tid2, wout2, bout2, rb, n)
